# baseline (speedup 1.0000x reference)
import functools

import jax
import jax.numpy as jnp
from jax import lax
from jax.experimental import pallas as pl
from jax.experimental.pallas import tpu as pltpu
from jax.experimental.pallas import tpu_sc as plsc

N = 100000
E = 1600000
G = 64

NC = 2
NS = 16
CHUNK = 128
K = -(-E // (NS * CHUNK))
E_PAD = K * NS * CHUNK
KSEG = 2
KS = K // KSEG
HN = N // NC
H = 50048
DUMMY = H - 1
RPT = H // NS
RB = 5000
NB = N // RB
RBH = 2000
NBH = N // RBH
NF = 100096
RPTF = NF // NS
KD = K // NC
NBUF = 4

_MESH = plsc.VectorSubcoreMesh(
    core_axis_name="c", subcore_axis_name="s", num_cores=NC, num_subcores=NS)
_SC_PARAMS = pltpu.CompilerParams(use_tc_tiling_on_sc=False)


def _make_deg_pass():
  @functools.partial(
      pl.kernel,
      out_type=jax.ShapeDtypeStruct((NC, NF), jnp.float32),
      mesh=_MESH,
      compiler_params=_SC_PARAMS,
      scratch_types=[
          pltpu.VMEM((KD, CHUNK), jnp.int32),
          pltpu.VMEM((CHUNK,), jnp.float32),
          pltpu.VMEM_SHARED((NF,), jnp.float32),
          [pltpu.SemaphoreType.DMA] * NBUF,
      ],
  )
  def deg_kernel(dst_hbm, ones_hbm, zeros_hbm, out_hbm,
                 dst_v, ones_v, acc, ssem):
    cid = lax.axis_index("c")
    sid = lax.axis_index("s")
    r0 = sid * RPTF
    pltpu.sync_copy(zeros_hbm, acc.at[pl.ds(r0, RPTF)])
    pltpu.sync_copy(ones_hbm, ones_v)
    pltpu.sync_copy(dst_hbm.at[cid, sid], dst_v)
    plsc.subcore_barrier()

    kd_main = (KD // NBUF) * NBUF

    def body(jj, carry):
      j0 = jj * NBUF
      for b in range(NBUF):
        pltpu.async_copy(ones_v, acc.at[dst_v.at[j0 + b]], ssem[b], add=True)
      for b in range(NBUF):
        pltpu.make_async_copy(
            ones_v, acc.at[dst_v.at[j0 + b]], ssem[b]).wait()
      return carry

    lax.fori_loop(0, kd_main // NBUF, body, 0)
    for j in range(kd_main, KD):
      pltpu.sync_copy(ones_v, acc.at[dst_v.at[j]], add=True)
    plsc.subcore_barrier()
    pltpu.sync_copy(acc.at[pl.ds(r0, RPTF)], out_hbm.at[cid, pl.ds(r0, RPTF)])

  return deg_kernel


def _make_agg_pass(D):
  @functools.partial(
      pl.kernel,
      out_type=jax.ShapeDtypeStruct((NC, H, D), jnp.float32),
      mesh=_MESH,
      compiler_params=_SC_PARAMS,
      scratch_types=[
          pltpu.VMEM((KS, CHUNK), jnp.int32),
          pltpu.VMEM((KS, CHUNK), jnp.int32),
          pltpu.VMEM((NBUF, CHUNK, D), jnp.float32),
          pltpu.VMEM_SHARED((H, D), jnp.float32),
          [pltpu.SemaphoreType.DMA] * NBUF,
          [pltpu.SemaphoreType.DMA] * NBUF,
      ],
  )
  def agg_kernel(g_hbm, src_hbm, dst_hbm, zeros_hbm, out_hbm,
                 src_v, dst_v, rows_v, acc, gsem, ssem):
    cid = lax.axis_index("c")
    sid = lax.axis_index("s")
    r0 = sid * RPT
    pltpu.sync_copy(zeros_hbm, acc.at[pl.ds(r0, RPT), :])
    plsc.subcore_barrier()

    ks_main = (KS // NBUF) * NBUF

    for seg in range(KSEG):
      pltpu.sync_copy(src_hbm.at[sid, pl.ds(seg * KS, KS)], src_v)
      pltpu.sync_copy(dst_hbm.at[cid, sid, pl.ds(seg * KS, KS)], dst_v)

      for b in range(NBUF):
        pltpu.async_copy(g_hbm.at[src_v.at[b]], rows_v.at[b], gsem[b])

      def body(jj, carry):
        j0 = jj * NBUF
        for b in range(NBUF):
          pltpu.make_async_copy(
              g_hbm.at[src_v.at[j0 + b]], rows_v.at[b], gsem[b]).wait()
          pltpu.async_copy(
              rows_v.at[b], acc.at[dst_v.at[j0 + b]], ssem[b], add=True)
        for b in range(NBUF):
          pltpu.make_async_copy(
              rows_v.at[b], acc.at[dst_v.at[j0 + b]], ssem[b]).wait()

          @pl.when(j0 + NBUF + b < KS)
          def _(b=b, j0=j0):
            pltpu.async_copy(
                g_hbm.at[src_v.at[j0 + NBUF + b]], rows_v.at[b], gsem[b])

        return carry

      lax.fori_loop(0, ks_main // NBUF, body, 0)
      for j in range(ks_main, KS):
        b = j - ks_main
        pltpu.make_async_copy(
            g_hbm.at[src_v.at[j]], rows_v.at[b], gsem[b]).wait()
        pltpu.sync_copy(rows_v.at[b], acc.at[dst_v.at[j]], add=True)

    plsc.subcore_barrier()
    pltpu.sync_copy(acc.at[pl.ds(r0, RPT), :],
                    out_hbm.at[cid, pl.ds(r0, RPT), :])

  return agg_kernel


def _tc_scale_kernel(deg0, deg1, x, dinv_o, g1_o):
  dinv = lax.rsqrt(deg0[...] + deg1[...] + 1.0)
  dinv_o[...] = dinv
  g1_o[...] = jnp.concatenate(
      [x[...] * dinv, jnp.zeros((RB, 5), jnp.float32)], axis=1)


def _tc_layer1_kernel(a1, g1, dinv, w1, b1, g2a_o, g2b_o):
  s = (a1[0] + g1[...]) * dinv[...]
  h = jnp.dot(s, w1[...], preferred_element_type=jnp.float32) + b1[...]
  g2 = jnp.maximum(h, 0.0) * dinv[...]
  g2a_o[...] = g2[:, :8]
  g2b_o[...] = g2[:, 8:]


def _tc_head_kernel(a2a, a2b, g2a, g2b, dinv, bat, w2, b2, f1w, f1b, f2w, f2b,
                    out, pool_acc, cnt_acc):
  i = pl.program_id(0)

  @pl.when(i == 0)
  def _():
    pool_acc[...] = jnp.zeros_like(pool_acc)
    cnt_acc[...] = jnp.zeros_like(cnt_acc)

  s = jnp.concatenate(
      [a2a[0] + g2a[...], a2b[0] + g2b[...]], axis=1) * dinv[...]
  h = jnp.maximum(
      jnp.dot(s, w2[...], preferred_element_type=jnp.float32) + b2[...], 0.0)
  gids = lax.broadcasted_iota(jnp.int32, (RBH, G), 1)
  oh = (bat[...] == gids).astype(jnp.float32)
  pool_acc[...] += lax.dot_general(
      oh, h, (((0,), (0,)), ((), ())), preferred_element_type=jnp.float32)
  cnt_acc[...] += lax.dot_general(
      oh, jnp.ones((RBH, 1), jnp.float32), (((0,), (0,)), ((), ())),
      preferred_element_type=jnp.float32)

  @pl.when(i == NBH - 1)
  def _():
    pooled = pool_acc[...] / jnp.maximum(cnt_acc[...], 1.0)
    o = jnp.dot(pooled, f1w[...], preferred_element_type=jnp.float32) + f1b[...]
    o = jnp.dot(o, f2w[...], preferred_element_type=jnp.float32) + f2b[...]
    out[...] = o


_deg_pass = _make_deg_pass()
_agg8_pass = _make_agg_pass(8)


def _row_spec(w):
  return pl.BlockSpec((RB, w), lambda i: (i, 0))


def _hrow_spec(w):
  return pl.BlockSpec((RBH, w), lambda i: (i, 0))


def _full_spec(shape):
  return pl.BlockSpec(shape, lambda i: tuple(0 for _ in shape))


def _agg_spec(rb):
  per = HN // rb
  return pl.BlockSpec((1, rb, 8), lambda i, per=per: (i // per, i % per, 0))


def kernel(x, edge_index, batch, W1, b1, W2, b2, fc1_w, fc1_b, fc2_w, fc2_b):
  src = edge_index[0].astype(jnp.int32)
  dst = edge_index[1].astype(jnp.int32)
  src_r = jnp.pad(src, (0, E_PAD - E)).reshape(NS, K, CHUNK)
  dst_p = jnp.pad(dst, (0, E_PAD - E), constant_values=N)
  dst0 = jnp.where(dst_p < HN, dst_p, DUMMY)
  dst1 = jnp.where((dst_p >= HN) & (dst_p < N), dst_p - HN, DUMMY)
  dst_r = jnp.stack([dst0, dst1]).reshape(NC, NS, K, CHUNK)

  bat_n = batch.astype(jnp.int32).reshape(N, 1)

  dst_full_r = dst_p.reshape(NC, NS, KD, CHUNK)

  ones_c = jnp.ones((CHUNK,), jnp.float32)
  z1 = jnp.zeros((RPTF,), jnp.float32)
  z8 = jnp.zeros((RPT, 8), jnp.float32)

  deg_p = _deg_pass(dst_full_r, ones_c, z1)

  dinv, g1 = pl.pallas_call(
      _tc_scale_kernel,
      grid=(NB,),
      in_specs=[_row_spec(1), _row_spec(1), _row_spec(3)],
      out_specs=[_row_spec(1), _row_spec(8)],
      out_shape=[jax.ShapeDtypeStruct((N, 1), jnp.float32),
                 jax.ShapeDtypeStruct((N, 8), jnp.float32)],
  )(deg_p[0, :N].reshape(N, 1), deg_p[1, :N].reshape(N, 1), x)

  a1 = _agg8_pass(g1, src_r, dst_r, z8)

  w1p = jnp.pad(W1, ((0, 5), (0, 0)))
  g2a, g2b = pl.pallas_call(
      _tc_layer1_kernel,
      grid=(NB,),
      in_specs=[_agg_spec(RB), _row_spec(8), _row_spec(1),
                _full_spec((8, 16)), _full_spec((1, 16))],
      out_specs=[_row_spec(8), _row_spec(8)],
      out_shape=[jax.ShapeDtypeStruct((N, 8), jnp.float32),
                 jax.ShapeDtypeStruct((N, 8), jnp.float32)],
  )(a1, g1, dinv, w1p, b1.reshape(1, 16))

  a2a = _agg8_pass(g2a, src_r, dst_r, z8)
  z8b = z8 + a2a[0, 0, :1] * 0.0
  a2b = _agg8_pass(g2b, src_r, dst_r, z8b)

  out = pl.pallas_call(
      _tc_head_kernel,
      grid=(NBH,),
      in_specs=[_agg_spec(RBH), _agg_spec(RBH), _hrow_spec(8), _hrow_spec(8),
                _hrow_spec(1), _hrow_spec(1),
                _full_spec((16, 64)), _full_spec((1, 64)),
                _full_spec((64, 32)), _full_spec((1, 32)),
                _full_spec((32, 2)), _full_spec((1, 2))],
      out_specs=_full_spec((G, 2)),
      out_shape=jax.ShapeDtypeStruct((G, 2), jnp.float32),
      scratch_shapes=[pltpu.VMEM((G, G), jnp.float32),
                      pltpu.VMEM((G, 1), jnp.float32)],
  )(a2a, a2b, g2a, g2b, dinv, bat_n,
    W2, b2.reshape(1, 64), fc1_w, fc1_b.reshape(1, 32),
    fc2_w, fc2_b.reshape(1, 2))

  return out


# speedup vs baseline: 1.0009x; 1.0009x over previous
import functools

import jax
import jax.numpy as jnp
from jax import lax
from jax.experimental import pallas as pl
from jax.experimental.pallas import tpu as pltpu
from jax.experimental.pallas import tpu_sc as plsc

N = 100000
E = 1600000
G = 64

NC = 2
NS = 16
CHUNK = 128
K = -(-E // (NS * CHUNK))
E_PAD = K * NS * CHUNK
KSEG = 2
KS = K // KSEG
HN = N // NC
H = 50048
DUMMY = H - 1
RPT = H // NS
RB = 5000
NB = N // RB
RBH = 2000
NBH = N // RBH
NF = 100096
RPTF = NF // NS
KD = K // NC
NBUF = 5

_MESH = plsc.VectorSubcoreMesh(
    core_axis_name="c", subcore_axis_name="s", num_cores=NC, num_subcores=NS)
_SC_PARAMS = pltpu.CompilerParams(use_tc_tiling_on_sc=False)


def _make_deg_pass():
  @functools.partial(
      pl.kernel,
      out_type=jax.ShapeDtypeStruct((NC, NF), jnp.float32),
      mesh=_MESH,
      compiler_params=_SC_PARAMS,
      scratch_types=[
          pltpu.VMEM((KD, CHUNK), jnp.int32),
          pltpu.VMEM((CHUNK,), jnp.float32),
          pltpu.VMEM_SHARED((NF,), jnp.float32),
          [pltpu.SemaphoreType.DMA] * NBUF,
      ],
  )
  def deg_kernel(dst_hbm, ones_hbm, zeros_hbm, out_hbm,
                 dst_v, ones_v, acc, ssem):
    cid = lax.axis_index("c")
    sid = lax.axis_index("s")
    r0 = sid * RPTF
    pltpu.sync_copy(zeros_hbm, acc.at[pl.ds(r0, RPTF)])
    pltpu.sync_copy(ones_hbm, ones_v)
    pltpu.sync_copy(dst_hbm.at[cid, sid], dst_v)
    plsc.subcore_barrier()

    kd_main = (KD // NBUF) * NBUF

    def body(jj, carry):
      j0 = jj * NBUF
      for b in range(NBUF):
        pltpu.async_copy(ones_v, acc.at[dst_v.at[j0 + b]], ssem[b], add=True)
      for b in range(NBUF):
        pltpu.make_async_copy(
            ones_v, acc.at[dst_v.at[j0 + b]], ssem[b]).wait()
      return carry

    lax.fori_loop(0, kd_main // NBUF, body, 0)
    for j in range(kd_main, KD):
      pltpu.sync_copy(ones_v, acc.at[dst_v.at[j]], add=True)
    plsc.subcore_barrier()
    pltpu.sync_copy(acc.at[pl.ds(r0, RPTF)], out_hbm.at[cid, pl.ds(r0, RPTF)])

  return deg_kernel


def _make_agg_pass(D):
  @functools.partial(
      pl.kernel,
      out_type=jax.ShapeDtypeStruct((NC, H, D), jnp.float32),
      mesh=_MESH,
      compiler_params=_SC_PARAMS,
      scratch_types=[
          pltpu.VMEM((KS, CHUNK), jnp.int32),
          pltpu.VMEM((KS, CHUNK), jnp.int32),
          pltpu.VMEM((NBUF, CHUNK, D), jnp.float32),
          pltpu.VMEM_SHARED((H, D), jnp.float32),
          [pltpu.SemaphoreType.DMA] * NBUF,
          [pltpu.SemaphoreType.DMA] * NBUF,
      ],
  )
  def agg_kernel(g_hbm, src_hbm, dst_hbm, zeros_hbm, out_hbm,
                 src_v, dst_v, rows_v, acc, gsem, ssem):
    cid = lax.axis_index("c")
    sid = lax.axis_index("s")
    r0 = sid * RPT
    pltpu.sync_copy(zeros_hbm, acc.at[pl.ds(r0, RPT), :])
    plsc.subcore_barrier()

    ks_main = (KS // NBUF) * NBUF

    for seg in range(KSEG):
      pltpu.sync_copy(src_hbm.at[sid, pl.ds(seg * KS, KS)], src_v)
      pltpu.sync_copy(dst_hbm.at[cid, sid, pl.ds(seg * KS, KS)], dst_v)

      for b in range(NBUF):
        pltpu.async_copy(g_hbm.at[src_v.at[b]], rows_v.at[b], gsem[b])

      def body(jj, carry):
        j0 = jj * NBUF
        for b in range(NBUF):
          pltpu.make_async_copy(
              g_hbm.at[src_v.at[j0 + b]], rows_v.at[b], gsem[b]).wait()
          pltpu.async_copy(
              rows_v.at[b], acc.at[dst_v.at[j0 + b]], ssem[b], add=True)
        for b in range(NBUF):
          pltpu.make_async_copy(
              rows_v.at[b], acc.at[dst_v.at[j0 + b]], ssem[b]).wait()

          @pl.when(j0 + NBUF + b < KS)
          def _(b=b, j0=j0):
            pltpu.async_copy(
                g_hbm.at[src_v.at[j0 + NBUF + b]], rows_v.at[b], gsem[b])

        return carry

      lax.fori_loop(0, ks_main // NBUF, body, 0)
      for j in range(ks_main, KS):
        b = j - ks_main
        pltpu.make_async_copy(
            g_hbm.at[src_v.at[j]], rows_v.at[b], gsem[b]).wait()
        pltpu.sync_copy(rows_v.at[b], acc.at[dst_v.at[j]], add=True)

    plsc.subcore_barrier()
    pltpu.sync_copy(acc.at[pl.ds(r0, RPT), :],
                    out_hbm.at[cid, pl.ds(r0, RPT), :])

  return agg_kernel


def _tc_scale_kernel(deg0, deg1, x, dinv_o, g1_o):
  dinv = lax.rsqrt(deg0[...] + deg1[...] + 1.0)
  dinv_o[...] = dinv
  g1_o[...] = jnp.concatenate(
      [x[...] * dinv, jnp.zeros((RB, 5), jnp.float32)], axis=1)


def _tc_layer1_kernel(a1, g1, dinv, w1, b1, g2a_o, g2b_o):
  s = (a1[0] + g1[...]) * dinv[...]
  h = jnp.dot(s, w1[...], preferred_element_type=jnp.float32) + b1[...]
  g2 = jnp.maximum(h, 0.0) * dinv[...]
  g2a_o[...] = g2[:, :8]
  g2b_o[...] = g2[:, 8:]


def _tc_head_kernel(a2a, a2b, g2a, g2b, dinv, bat, w2, b2, f1w, f1b, f2w, f2b,
                    out, pool_acc, cnt_acc):
  i = pl.program_id(0)

  @pl.when(i == 0)
  def _():
    pool_acc[...] = jnp.zeros_like(pool_acc)
    cnt_acc[...] = jnp.zeros_like(cnt_acc)

  s = jnp.concatenate(
      [a2a[0] + g2a[...], a2b[0] + g2b[...]], axis=1) * dinv[...]
  h = jnp.maximum(
      jnp.dot(s, w2[...], preferred_element_type=jnp.float32) + b2[...], 0.0)
  gids = lax.broadcasted_iota(jnp.int32, (RBH, G), 1)
  oh = (bat[...] == gids).astype(jnp.float32)
  pool_acc[...] += lax.dot_general(
      oh, h, (((0,), (0,)), ((), ())), preferred_element_type=jnp.float32)
  cnt_acc[...] += lax.dot_general(
      oh, jnp.ones((RBH, 1), jnp.float32), (((0,), (0,)), ((), ())),
      preferred_element_type=jnp.float32)

  @pl.when(i == NBH - 1)
  def _():
    pooled = pool_acc[...] / jnp.maximum(cnt_acc[...], 1.0)
    o = jnp.dot(pooled, f1w[...], preferred_element_type=jnp.float32) + f1b[...]
    o = jnp.dot(o, f2w[...], preferred_element_type=jnp.float32) + f2b[...]
    out[...] = o


_deg_pass = _make_deg_pass()
_agg8_pass = _make_agg_pass(8)


def _row_spec(w):
  return pl.BlockSpec((RB, w), lambda i: (i, 0))


def _hrow_spec(w):
  return pl.BlockSpec((RBH, w), lambda i: (i, 0))


def _full_spec(shape):
  return pl.BlockSpec(shape, lambda i: tuple(0 for _ in shape))


def _agg_spec(rb):
  per = HN // rb
  return pl.BlockSpec((1, rb, 8), lambda i, per=per: (i // per, i % per, 0))


def kernel(x, edge_index, batch, W1, b1, W2, b2, fc1_w, fc1_b, fc2_w, fc2_b):
  src = edge_index[0].astype(jnp.int32)
  dst = edge_index[1].astype(jnp.int32)
  src_r = jnp.pad(src, (0, E_PAD - E)).reshape(NS, K, CHUNK)
  dst_p = jnp.pad(dst, (0, E_PAD - E), constant_values=N)
  dst0 = jnp.where(dst_p < HN, dst_p, DUMMY)
  dst1 = jnp.where((dst_p >= HN) & (dst_p < N), dst_p - HN, DUMMY)
  dst_r = jnp.stack([dst0, dst1]).reshape(NC, NS, K, CHUNK)

  bat_n = batch.astype(jnp.int32).reshape(N, 1)

  dst_full_r = dst_p.reshape(NC, NS, KD, CHUNK)

  ones_c = jnp.ones((CHUNK,), jnp.float32)
  z1 = jnp.zeros((RPTF,), jnp.float32)
  z8 = jnp.zeros((RPT, 8), jnp.float32)

  deg_p = _deg_pass(dst_full_r, ones_c, z1)

  dinv, g1 = pl.pallas_call(
      _tc_scale_kernel,
      grid=(NB,),
      in_specs=[_row_spec(1), _row_spec(1), _row_spec(3)],
      out_specs=[_row_spec(1), _row_spec(8)],
      out_shape=[jax.ShapeDtypeStruct((N, 1), jnp.float32),
                 jax.ShapeDtypeStruct((N, 8), jnp.float32)],
  )(deg_p[0, :N].reshape(N, 1), deg_p[1, :N].reshape(N, 1), x)

  a1 = _agg8_pass(g1, src_r, dst_r, z8)

  w1p = jnp.pad(W1, ((0, 5), (0, 0)))
  g2a, g2b = pl.pallas_call(
      _tc_layer1_kernel,
      grid=(NB,),
      in_specs=[_agg_spec(RB), _row_spec(8), _row_spec(1),
                _full_spec((8, 16)), _full_spec((1, 16))],
      out_specs=[_row_spec(8), _row_spec(8)],
      out_shape=[jax.ShapeDtypeStruct((N, 8), jnp.float32),
                 jax.ShapeDtypeStruct((N, 8), jnp.float32)],
  )(a1, g1, dinv, w1p, b1.reshape(1, 16))

  a2a = _agg8_pass(g2a, src_r, dst_r, z8)
  z8b = z8 + a2a[0, 0, :1] * 0.0
  a2b = _agg8_pass(g2b, src_r, dst_r, z8b)

  out = pl.pallas_call(
      _tc_head_kernel,
      grid=(NBH,),
      in_specs=[_agg_spec(RBH), _agg_spec(RBH), _hrow_spec(8), _hrow_spec(8),
                _hrow_spec(1), _hrow_spec(1),
                _full_spec((16, 64)), _full_spec((1, 64)),
                _full_spec((64, 32)), _full_spec((1, 32)),
                _full_spec((32, 2)), _full_spec((1, 2))],
      out_specs=_full_spec((G, 2)),
      out_shape=jax.ShapeDtypeStruct((G, 2), jnp.float32),
      scratch_shapes=[pltpu.VMEM((G, G), jnp.float32),
                      pltpu.VMEM((G, 1), jnp.float32)],
  )(a2a, a2b, g2a, g2b, dinv, bat_n,
    W2, b2.reshape(1, 64), fc1_w, fc1_b.reshape(1, 32),
    fc2_w, fc2_b.reshape(1, 2))

  return out
